# baseline (speedup 1.0000x reference)
import functools

import jax
import jax.numpy as jnp
from jax import lax
from jax.experimental import pallas as pl
from jax.experimental.pallas import tpu as pltpu
from jax.experimental.pallas import tpu_sc as plsc

DIM = 128
BATCH = 4096
HIST = 50
HPAD = 56
NCORES = 2
NSUB = 16
NW = NCORES * NSUB
BPW = BATCH // NW
NBUF = 8
HSPLIT = 24
LANES = 16

_SELU_ALPHA = 1.6732632423543772
_SELU_SCALE = 1.0507009873554805


def _sc_pool(idx_flat, table):
    mesh = plsc.VectorSubcoreMesh(
        core_axis_name="c", subcore_axis_name="s",
        num_cores=NCORES, num_subcores=NSUB,
    )

    @functools.partial(
        pl.kernel,
        out_type=jax.ShapeDtypeStruct((BATCH, DIM), jnp.float32),
        mesh=mesh,
        scratch_types=[
            pltpu.VMEM((BPW, HPAD), jnp.int32),
            pltpu.VMEM((BPW, DIM), jnp.float32),
            pltpu.VMEM((NBUF, HIST, DIM), jnp.float32),
            pltpu.SemaphoreType.DMA((NBUF,)),
            pltpu.SemaphoreType.DMA((NBUF,)),
        ],
    )
    def pool(idx_hbm, table_hbm, out_hbm, idx_v, outbuf, bufs, sems_a, sems_b):
        wid = lax.axis_index("c") * NSUB + lax.axis_index("s")
        base = wid * BPW
        pltpu.sync_copy(idx_hbm.at[pl.ds(base, BPW), :], idx_v)

        def dma_a(row, slot):
            return pltpu.make_async_copy(
                table_hbm.at[idx_v.at[row, pl.ds(0, HSPLIT)]],
                bufs.at[slot, pl.ds(0, HSPLIT)], sems_a.at[slot],
            )

        def dma_b(row, slot):
            return pltpu.make_async_copy(
                table_hbm.at[idx_v.at[row, pl.ds(HSPLIT, HIST - HSPLIT)]],
                bufs.at[slot, pl.ds(HSPLIT, HIST - HSPLIT)], sems_b.at[slot],
            )

        def start(row, carry):
            slot = lax.rem(row, NBUF)
            dma_a(row, slot).start()
            dma_b(row, slot).start()
            return carry

        lax.fori_loop(0, NBUF, start, 0)

        def body(row, carry):
            slot = lax.rem(row, NBUF)
            dma_a(row, slot).wait()
            dma_b(row, slot).wait()
            accs = tuple(bufs[slot, 0, pl.ds(d * LANES, LANES)] for d in range(8))

            def inner(h, a8):
                return tuple(
                    a + bufs[slot, h, pl.ds(d * LANES, LANES)]
                    for d, a in enumerate(a8)
                )

            accs = lax.fori_loop(1, HIST, inner, accs, unroll=7)
            for d in range(8):
                outbuf[row, pl.ds(d * LANES, LANES)] = accs[d]

            @pl.when(row + NBUF < BPW)
            def _():
                dma_a(row + NBUF, slot).start()
                dma_b(row + NBUF, slot).start()

            return carry

        lax.fori_loop(0, BPW, body, 0)
        pltpu.sync_copy(outbuf, out_hbm.at[pl.ds(base, BPW)])

    return pool(idx_flat, table)


def _selu_linear(x, w, b2):
    def body(x_ref, w_ref, b_ref, o_ref):
        v = x_ref[...]
        v = _SELU_SCALE * jnp.where(v > 0, v, _SELU_ALPHA * (jnp.exp(v) - 1.0))
        o_ref[...] = (
            lax.dot_general(
                v, w_ref[...],
                dimension_numbers=(((1,), (1,)), ((), ())),
                preferred_element_type=jnp.float32,
            )
            + b_ref[...]
        )

    return pl.pallas_call(
        body,
        out_shape=jax.ShapeDtypeStruct((BATCH, DIM), jnp.float32),
    )(x, w, b2)


def kernel(input_text, table, W, b):
    idx = input_text.reshape(BATCH, HIST).astype(jnp.int32)
    idx = jnp.pad(idx, ((0, 0), (0, HPAD - HIST)))
    pooled = _sc_pool(idx, table)
    return _selu_linear(pooled, W, b.reshape(1, DIM))


# speedup vs baseline: 1.1551x; 1.1551x over previous
import functools

import jax
import jax.numpy as jnp
from jax import lax
from jax.experimental import pallas as pl
from jax.experimental.pallas import tpu as pltpu
from jax.experimental.pallas import tpu_sc as plsc

DIM = 128
BATCH = 4096
HIST = 50
HPAD = 56
NCORES = 2
NSUB = 16
NW = NCORES * NSUB
BPW = BATCH // NW
NBUF = 8
LANES = 16

_SELU_ALPHA = 1.6732632423543772
_SELU_SCALE = 1.0507009873554805


def _sc_pool(idx_flat, table):
    mesh = plsc.VectorSubcoreMesh(
        core_axis_name="c", subcore_axis_name="s",
        num_cores=NCORES, num_subcores=NSUB,
    )

    @functools.partial(
        pl.kernel,
        out_type=jax.ShapeDtypeStruct((BATCH, DIM), jnp.float32),
        mesh=mesh,
        scratch_types=[
            pltpu.VMEM((BPW, HPAD), jnp.int32),
            pltpu.VMEM((BPW, DIM), jnp.float32),
            pltpu.VMEM((NBUF, HIST, DIM), jnp.float32),
            pltpu.SemaphoreType.DMA((NBUF,)),
        ],
    )
    def pool(idx_hbm, table_hbm, out_hbm, idx_v, outbuf, bufs, sems):
        wid = lax.axis_index("c") * NSUB + lax.axis_index("s")
        base = wid * BPW
        pltpu.sync_copy(idx_hbm.at[pl.ds(base, BPW), :], idx_v)

        def dma(row, slot):
            return pltpu.make_async_copy(
                table_hbm.at[idx_v.at[row, pl.ds(0, HIST)]],
                bufs.at[slot], sems.at[slot],
            )

        def start(row, carry):
            dma(row, lax.rem(row, NBUF)).start()
            return carry

        lax.fori_loop(0, NBUF, start, 0)

        def body(row, carry):
            slot = lax.rem(row, NBUF)
            dma(row, slot).wait()
            accs = tuple(bufs[slot, 0, pl.ds(d * LANES, LANES)] for d in range(8))

            def inner(h, a8):
                return tuple(
                    a + bufs[slot, h, pl.ds(d * LANES, LANES)]
                    for d, a in enumerate(a8)
                )

            accs = lax.fori_loop(1, HIST, inner, accs, unroll=7)
            for d in range(8):
                outbuf[row, pl.ds(d * LANES, LANES)] = accs[d]

            @pl.when(row + NBUF < BPW)
            def _():
                dma(row + NBUF, slot).start()

            return carry

        lax.fori_loop(0, BPW, body, 0)
        pltpu.sync_copy(outbuf, out_hbm.at[pl.ds(base, BPW)])

    return pool(idx_flat, table)


def _selu_linear(x, w, b2):
    def body(x_ref, w_ref, b_ref, o_ref):
        v = x_ref[...]
        v = _SELU_SCALE * jnp.where(v > 0, v, _SELU_ALPHA * (jnp.exp(v) - 1.0))
        o_ref[...] = (
            lax.dot_general(
                v, w_ref[...],
                dimension_numbers=(((1,), (1,)), ((), ())),
                preferred_element_type=jnp.float32,
            )
            + b_ref[...]
        )

    return pl.pallas_call(
        body,
        out_shape=jax.ShapeDtypeStruct((BATCH, DIM), jnp.float32),
    )(x, w, b2)


def kernel(input_text, table, W, b):
    idx = input_text.reshape(BATCH, HIST).astype(jnp.int32)
    idx = jnp.pad(idx, ((0, 0), (0, HPAD - HIST)))
    pooled = _sc_pool(idx, table)
    return _selu_linear(pooled, W, b.reshape(1, DIM))
